# Initial kernel scaffold; baseline (speedup 1.0000x reference)
import jax
import jax.numpy as jnp
from jax.experimental import pallas as pl


def kernel(x, gamma1, gamma2, Wa, ba, Wb, Wc1, Wc2, Wc3, Wmp, Wm1, bm1, Wm2, bm2):
    raise NotImplementedError("write your pallas kernel here")


# speedup vs baseline: 1.4471x; 1.4471x over previous
import functools
import math

import jax
import jax.numpy as jnp
from jax.experimental import pallas as pl
from jax.experimental.pallas import tpu as pltpu

B, S, D = 4, 8192, 768
NF = D // 4
K = math.ceil(0.1 * S)
KPAD = 832
TILE_S = 1024
NT = S // TILE_S
SQRT_D = D ** 0.5
NH = 4
HB = (4 * D) // NH
EXTH = KPAD + 40
NVALID = K + 1


def _gelu(x):
    return 0.5 * x * (1.0 + jax.lax.erf(x * (2.0 ** -0.5)))


def _rms_rows(xb, g):
    n = jnp.sqrt(jnp.sum(xb * xb, axis=1, keepdims=True))
    n = jnp.maximum(n, 1e-12)
    return xb / n * (g * SQRT_D)


def _score_kernel(x_ref, g1_ref, wa_ref, ba_ref, score_ref, tsum_ref):
    t = pl.program_id(1)
    xb = x_ref[0]
    xn = _rms_rows(xb, g1_ref[0])
    logit = jnp.dot(xn, wa_ref[...], preferred_element_type=jnp.float32)
    s = jax.nn.sigmoid(logit + ba_ref[0, 0])
    score_ref[0, 0, :] = s[:, 0]
    part = jnp.dot(s.T, xn, preferred_element_type=jnp.float32)

    @pl.when(t == 0)
    def _():
        tsum_ref[...] = jnp.zeros_like(tsum_ref)

    tsum_ref[0] += part


def _gather_kernel(idx_ref, x_ref, score_ref, g1_ref, tsum_ref, xcat_ref):
    t = pl.program_id(1)

    @pl.when(t == 0)
    def _():
        xcat_ref[...] = jnp.zeros_like(xcat_ref)

    xb = x_ref[0]
    xn = _rms_rows(xb, g1_ref[0])
    s = score_ref[0, 0, :]
    xs = xn * s[:, None]
    idx = idx_ref[0, 0, :]
    local = idx - t * TILE_S
    col = jax.lax.broadcasted_iota(jnp.int32, (KPAD, TILE_S), 1)
    p = (local[:, None] == col).astype(jnp.float32)
    xcat_ref[0] += jax.lax.dot(p, xs,
                               precision=jax.lax.Precision.HIGHEST,
                               preferred_element_type=jnp.float32)

    @pl.when(t == NT - 1)
    def _():
        acc = xcat_ref[0]
        topk_sum = jnp.sum(acc, axis=0, keepdims=True)
        extra = tsum_ref[0] - topk_sum
        row = jax.lax.broadcasted_iota(jnp.int32, (KPAD, 1), 0)
        xcat_ref[0] = jnp.where(row == K, extra, acc)


def _moe_kernel(xcat_ref, g2_ref, wm1_ref, bm1_ref, wm2_ref, bm2_ref,
                moe_ref, xn2_scr):
    h = pl.program_id(1)

    @pl.when(h == 0)
    def _():
        xn2_scr[...] = _rms_rows(xcat_ref[0], g2_ref[0])
        moe_ref[0] = jnp.broadcast_to(bm2_ref[0], (KPAD, D))

    xn2 = xn2_scr[...]
    hid = _gelu(jnp.dot(xn2, wm1_ref[...],
                        preferred_element_type=jnp.float32) + bm1_ref[0])
    moe_ref[0] += jnp.dot(hid, wm2_ref[...],
                          preferred_element_type=jnp.float32)


def _incep_kernel(xcat_ref, moe_ref, g2_ref, wb_ref, wc1_ref, wc2_ref,
                  wc3_ref, wmp_ref, wa_ref, ba_ref, y_ref, attn_ref, ext_scr):
    xc = xcat_ref[0]
    xn2 = _rms_rows(xc, g2_ref[0])
    xb = jnp.dot(xn2, wb_ref[...], preferred_element_type=jnp.float32)

    ext_scr[...] = jnp.zeros_like(ext_scr)
    ext_scr[19:19 + KPAD, :] = xb

    def conv(w_ref, ntap, off):
        acc = jnp.zeros((KPAD, NF), jnp.float32)
        for t in range(ntap):
            sl = ext_scr[off + t: off + t + KPAD, :]
            acc += jnp.dot(sl, w_ref[t], preferred_element_type=jnp.float32)
        return acc

    c1 = conv(wc1_ref, 39, 0)
    c2 = conv(wc2_ref, 19, 10)
    c3 = conv(wc3_ref, 9, 15)

    neg = jnp.float32(-1e30)
    rowi = jax.lax.broadcasted_iota(jnp.int32, (KPAD, 1), 0)
    xm = jnp.where(rowi < NVALID, xn2, neg)
    up = jnp.concatenate([xm[1:], jnp.full((1, D), neg)], axis=0)
    down = jnp.concatenate([jnp.full((1, D), neg), xm[:-1]], axis=0)
    pooled = jnp.maximum(jnp.maximum(up, down), xm)
    pooled = jnp.where(rowi < NVALID, pooled, 0.0)
    mpc = jnp.dot(pooled, wmp_ref[...], preferred_element_type=jnp.float32)

    cat = jnp.concatenate([c1, c2, c3, mpc], axis=1)
    cat = jnp.where(rowi < NVALID, cat, 0.0)
    mean = jnp.sum(cat, axis=0, keepdims=True) / NVALID
    ex2 = jnp.sum(cat * cat, axis=0, keepdims=True) / NVALID
    var = ex2 - mean * mean
    normed = (cat - mean) / jnp.sqrt(var + 1e-5)
    incep = _gelu(normed)

    out = xc + moe_ref[0] + incep
    y_ref[0] = _gelu(out)
    logit = jnp.dot(out, wa_ref[...], preferred_element_type=jnp.float32)
    attn_ref[0, 0, :] = jax.nn.sigmoid(logit + ba_ref[0, 0])[:, 0]


@jax.jit
def kernel(x, gamma1, gamma2, Wa, ba, Wb, Wc1, Wc2, Wc3, Wmp, Wm1, bm1,
           Wm2, bm2):
    g1 = gamma1.reshape(1, D)
    g2 = gamma2.reshape(1, D)
    ba2 = ba.reshape(1, 1)

    score, tsum = pl.pallas_call(
        _score_kernel,
        grid=(B, NT),
        in_specs=[
            pl.BlockSpec((1, TILE_S, D), lambda b, t: (b, t, 0)),
            pl.BlockSpec((1, D), lambda b, t: (0, 0)),
            pl.BlockSpec((D, 1), lambda b, t: (0, 0)),
            pl.BlockSpec((1, 1), lambda b, t: (0, 0)),
        ],
        out_specs=[
            pl.BlockSpec((1, 1, TILE_S), lambda b, t: (b, 0, t)),
            pl.BlockSpec((1, 1, D), lambda b, t: (b, 0, 0)),
        ],
        out_shape=[
            jax.ShapeDtypeStruct((B, 1, S), jnp.float32),
            jax.ShapeDtypeStruct((B, 1, D), jnp.float32),
        ],
    )(x, g1, Wa, ba2)

    _, idx = jax.lax.top_k(score[:, 0, :], K)
    sorted_idx = jnp.sort(idx, axis=1).astype(jnp.int32)
    idx_pad = jnp.concatenate(
        [sorted_idx, jnp.full((B, KPAD - K), -1, jnp.int32)],
        axis=1).reshape(B, 1, KPAD)

    xcat = pl.pallas_call(
        _gather_kernel,
        grid=(B, NT),
        in_specs=[
            pl.BlockSpec((1, 1, KPAD), lambda b, t: (b, 0, 0)),
            pl.BlockSpec((1, TILE_S, D), lambda b, t: (b, t, 0)),
            pl.BlockSpec((1, 1, TILE_S), lambda b, t: (b, 0, t)),
            pl.BlockSpec((1, D), lambda b, t: (0, 0)),
            pl.BlockSpec((1, 1, D), lambda b, t: (b, 0, 0)),
        ],
        out_specs=pl.BlockSpec((1, KPAD, D), lambda b, t: (b, 0, 0)),
        out_shape=jax.ShapeDtypeStruct((B, KPAD, D), jnp.float32),
    )(idx_pad, x, score, g1, tsum)

    bm1r = bm1.reshape(NH, 1, HB)
    bm2r = bm2.reshape(1, D)
    moe = pl.pallas_call(
        _moe_kernel,
        grid=(B, NH),
        in_specs=[
            pl.BlockSpec((1, KPAD, D), lambda b, h: (b, 0, 0)),
            pl.BlockSpec((1, D), lambda b, h: (0, 0)),
            pl.BlockSpec((D, HB), lambda b, h: (0, h)),
            pl.BlockSpec((1, 1, HB), lambda b, h: (h, 0, 0)),
            pl.BlockSpec((HB, D), lambda b, h: (h, 0)),
            pl.BlockSpec((1, D), lambda b, h: (0, 0)),
        ],
        out_specs=pl.BlockSpec((1, KPAD, D), lambda b, h: (b, 0, 0)),
        out_shape=jax.ShapeDtypeStruct((B, KPAD, D), jnp.float32),
        scratch_shapes=[pltpu.VMEM((KPAD, D), jnp.float32)],
    )(xcat, g2, Wm1, bm1r, Wm2, bm2r)

    wb_r = Wb[:, :, 0].T
    wmp_r = Wmp[:, :, 0].T
    wc1_r = jnp.transpose(Wc1, (2, 1, 0))
    wc2_r = jnp.transpose(Wc2, (2, 1, 0))
    wc3_r = jnp.transpose(Wc3, (2, 1, 0))

    y, attn = pl.pallas_call(
        _incep_kernel,
        grid=(B,),
        in_specs=[
            pl.BlockSpec((1, KPAD, D), lambda b: (b, 0, 0)),
            pl.BlockSpec((1, KPAD, D), lambda b: (b, 0, 0)),
            pl.BlockSpec((1, D), lambda b: (0, 0)),
            pl.BlockSpec((D, NF), lambda b: (0, 0)),
            pl.BlockSpec((39, NF, NF), lambda b: (0, 0, 0)),
            pl.BlockSpec((19, NF, NF), lambda b: (0, 0, 0)),
            pl.BlockSpec((9, NF, NF), lambda b: (0, 0, 0)),
            pl.BlockSpec((D, NF), lambda b: (0, 0)),
            pl.BlockSpec((D, 1), lambda b: (0, 0)),
            pl.BlockSpec((1, 1), lambda b: (0, 0)),
        ],
        out_specs=[
            pl.BlockSpec((1, KPAD, D), lambda b: (b, 0, 0)),
            pl.BlockSpec((1, 1, KPAD), lambda b: (b, 0, 0)),
        ],
        out_shape=[
            jax.ShapeDtypeStruct((B, KPAD, D), jnp.float32),
            jax.ShapeDtypeStruct((B, 1, KPAD), jnp.float32),
        ],
        scratch_shapes=[pltpu.VMEM((EXTH, NF), jnp.float32)],
    )(xcat, moe, g2, wb_r, wc1_r, wc2_r, wc3_r, wmp_r, Wa, ba2)

    yout = y[:, :NVALID, :]
    end_attn = attn[:, 0, :NVALID, None]
    return yout, jnp.float32(0.0), end_attn, sorted_idx
